# baseline (speedup 1.0000x reference)
import functools

import jax
import jax.numpy as jnp
from jax import lax
from jax.experimental import pallas as pl
from jax.experimental.pallas import tpu as pltpu
from jax.experimental.pallas import tpu_sc as plsc

_TOKENS = 409600
_PARTS = 4
_PTOK = _TOKENS // _PARTS
_D = 64
_USERS = 4096
_CW = 16
_NC = 2
_NS = 16
_NW = _NC * _NS
_TPW = _PTOK // _NW
_SUB = 128
_NSUB = 5
_CHUNK = _SUB * _NSUB
_NCH = _TPW // _CHUNK
_RPT = _USERS // _NS


def _sc_segment_sums(x_part, idx_part):
  mesh = plsc.VectorSubcoreMesh(core_axis_name="c", subcore_axis_name="s",
                                num_cores=_NC, num_subcores=_NS)

  scratch = ([pltpu.VMEM_SHARED((_USERS, _D), jnp.float32),
              pltpu.VMEM_SHARED((_USERS, _CW), jnp.float32)]
             + [pltpu.VMEM((_SUB, _D), jnp.float32) for _ in range(_NSUB)]
             + [pltpu.VMEM((_SUB,), jnp.int32) for _ in range(_NSUB)]
             + [pltpu.VMEM((_SUB, _CW), jnp.float32)]
             + [pltpu.SemaphoreType.DMA for _ in range(4 * _NSUB)])

  @functools.partial(
      pl.kernel,
      out_type=(jax.ShapeDtypeStruct((_NC, _USERS, _D), jnp.float32),
                jax.ShapeDtypeStruct((_NC, _USERS, _CW), jnp.float32)),
      mesh=mesh,
      scratch_types=scratch,
  )
  def run(x_hbm, idx_hbm, sums_hbm, cnts_hbm, acc, accc, *bufs):
    xb = bufs[:_NSUB]
    ib = bufs[_NSUB:2 * _NSUB]
    ones = bufs[2 * _NSUB]
    sems = bufs[2 * _NSUB + 1:]
    sxx = sems[:_NSUB]
    six = sems[_NSUB:2 * _NSUB]
    ssx = sems[2 * _NSUB:3 * _NSUB]
    ssc = sems[3 * _NSUB:]
    c = lax.axis_index("c")
    s = lax.axis_index("s")
    base = (c * _NS + s) * _TPW

    zf = jnp.zeros((16,), jnp.float32)
    onesv = jnp.full((16,), 1.0, jnp.float32)

    def zrow(r, carry):
      for k in range(_D // 16):
        xb[0][r, pl.ds(16 * k, 16)] = zf
      ones[r, pl.ds(0, 16)] = onesv
      return carry
    lax.fori_loop(0, _SUB, zrow, 0)
    for r in range(_RPT // _SUB):
      pltpu.sync_copy(xb[0], acc.at[pl.ds(s * _RPT + r * _SUB, _SUB)])
    def zcrow(r, carry):
      ones[r, pl.ds(0, 16)] = zf
      return carry
    lax.fori_loop(0, _SUB, zcrow, 0)
    for r in range(_RPT // _SUB):
      pltpu.sync_copy(ones, accc.at[pl.ds(s * _RPT + r * _SUB, _SUB)])
    def orow(r, carry):
      ones[r, pl.ds(0, 16)] = onesv
      return carry
    lax.fori_loop(0, _SUB, orow, 0)
    plsc.subcore_barrier()

    def body(i, carry):
      t0 = base + i * _CHUNK
      ins = []
      for j in range(_NSUB):
        @pl.when(i > 0)
        def _(j=j):
          pltpu.make_async_copy(xb[j], acc.at[ib[j]], ssx[j]).wait()
          pltpu.make_async_copy(ones, accc.at[ib[j]], ssc[j]).wait()
        o = t0 + j * _SUB
        hi = pltpu.async_copy(idx_hbm.at[pl.ds(o, _SUB)], ib[j], six[j])
        hx = pltpu.async_copy(x_hbm.at[pl.ds(o, _SUB)], xb[j], sxx[j])
        ins.append((hi, hx))
      for j in range(_NSUB):
        hi, hx = ins[j]
        hi.wait()
        hx.wait()
        pltpu.async_copy(xb[j], acc.at[ib[j]], ssx[j], add=True)
        pltpu.async_copy(ones, accc.at[ib[j]], ssc[j], add=True)
      return carry

    lax.fori_loop(0, _NCH, body, 0)
    for j in range(_NSUB):
      pltpu.make_async_copy(xb[j], acc.at[ib[j]], ssx[j]).wait()
      pltpu.make_async_copy(ones, accc.at[ib[j]], ssc[j]).wait()
    plsc.subcore_barrier()
    pltpu.sync_copy(acc.at[pl.ds(s * _RPT, _RPT)],
                    sums_hbm.at[c, pl.ds(s * _RPT, _RPT)])
    pltpu.sync_copy(accc.at[pl.ds(s * _RPT, _RPT)],
                    cnts_hbm.at[c, pl.ds(s * _RPT, _RPT)])

  return run(x_part, idx_part)


def _finalize(sums, cnts):
  def body(p_ref, q_ref, o_ref):
    p = p_ref[...]
    q = q_ref[...]
    tot = p[0, 0]
    cnt = q[0, 0, :, 0:1]
    for k in range(_PARTS):
      for i in range(_NC):
        if k == 0 and i == 0:
          continue
        tot = tot + p[k, i]
        cnt = cnt + q[k, i, :, 0:1]
    val = tot / cnt
    o_ref[...] = jnp.concatenate(
        [val, jnp.zeros((_USERS, 1), jnp.float32)], axis=1)

  return pl.pallas_call(
      body,
      out_shape=jax.ShapeDtypeStruct((_USERS, _D + 1), jnp.float32),
  )(sums, cnts)


@jax.jit
def kernel(x_hist, batch_hist):
  idx = batch_hist.astype(jnp.int32)
  parts_s = []
  parts_c = []
  for k in range(_PARTS):
    s, cpart = _sc_segment_sums(x_hist[k * _PTOK:(k + 1) * _PTOK],
                                idx[k * _PTOK:(k + 1) * _PTOK])
    parts_s.append(s)
    parts_c.append(cpart)
  return _finalize(jnp.stack(parts_s), jnp.stack(parts_c))


# speedup vs baseline: 1.0463x; 1.0463x over previous
import functools

import jax
import jax.numpy as jnp
from jax import lax
from jax.experimental import pallas as pl
from jax.experimental.pallas import tpu as pltpu
from jax.experimental.pallas import tpu_sc as plsc

_TOKENS = 409600
_PARTS = 4
_PTOK = _TOKENS // _PARTS
_D = 64
_USERS = 4096
_CW = 16
_NC = 2
_NS = 16
_NW = _NC * _NS
_TPW = _PTOK // _NW
_SUB = 128
_NSUB = 5
_CHUNK = _SUB * _NSUB
_NCH = _TPW // _CHUNK
_RPT = _USERS // _NS
_ITPW = _TOKENS // _NW
_INCH = _ITPW // _CHUNK


def _sc_counts(idx_full):
  mesh = plsc.VectorSubcoreMesh(core_axis_name="c", subcore_axis_name="s",
                                num_cores=_NC, num_subcores=_NS)
  scratch = ([pltpu.VMEM_SHARED((_USERS, _CW), jnp.float32)]
             + [pltpu.VMEM((_SUB,), jnp.int32) for _ in range(_NSUB)]
             + [pltpu.VMEM((_SUB, _CW), jnp.float32)]
             + [pltpu.SemaphoreType.DMA for _ in range(2 * _NSUB)])

  @functools.partial(
      pl.kernel,
      out_type=jax.ShapeDtypeStruct((_NC, _USERS, _CW), jnp.float32),
      mesh=mesh,
      scratch_types=scratch,
  )
  def run(idx_hbm, cnts_hbm, accc, *bufs):
    ib = bufs[:_NSUB]
    ones = bufs[_NSUB]
    sems = bufs[_NSUB + 1:]
    six = sems[:_NSUB]
    ssc = sems[_NSUB:]
    c = lax.axis_index("c")
    s = lax.axis_index("s")
    base = (c * _NS + s) * _ITPW

    zf = jnp.zeros((16,), jnp.float32)
    onesv = jnp.full((16,), 1.0, jnp.float32)

    def zcrow(r, carry):
      ones[r, pl.ds(0, 16)] = zf
      return carry
    lax.fori_loop(0, _SUB, zcrow, 0)
    for r in range(_RPT // _SUB):
      pltpu.sync_copy(ones, accc.at[pl.ds(s * _RPT + r * _SUB, _SUB)])
    def orow(r, carry):
      ones[r, pl.ds(0, 16)] = onesv
      return carry
    lax.fori_loop(0, _SUB, orow, 0)
    plsc.subcore_barrier()

    def body(i, carry):
      t0 = base + i * _CHUNK
      ins = []
      for j in range(_NSUB):
        @pl.when(i > 0)
        def _(j=j):
          pltpu.make_async_copy(ones, accc.at[ib[j]], ssc[j]).wait()
        ins.append(pltpu.async_copy(idx_hbm.at[pl.ds(t0 + j * _SUB, _SUB)],
                                    ib[j], six[j]))
      for j in range(_NSUB):
        ins[j].wait()
        pltpu.async_copy(ones, accc.at[ib[j]], ssc[j], add=True)
      return carry

    lax.fori_loop(0, _INCH, body, 0)
    for j in range(_NSUB):
      pltpu.make_async_copy(ones, accc.at[ib[j]], ssc[j]).wait()
    plsc.subcore_barrier()
    pltpu.sync_copy(accc.at[pl.ds(s * _RPT, _RPT)],
                    cnts_hbm.at[c, pl.ds(s * _RPT, _RPT)])

  return run(idx_full)


def _sc_part_sums(x_part, idx_full, part):
  mesh = plsc.VectorSubcoreMesh(core_axis_name="c", subcore_axis_name="s",
                                num_cores=_NC, num_subcores=_NS)

  scratch = ([pltpu.VMEM_SHARED((_USERS, _D), jnp.float32)]
             + [pltpu.VMEM((_SUB, _D), jnp.float32) for _ in range(_NSUB)]
             + [pltpu.VMEM((_SUB,), jnp.int32) for _ in range(_NSUB)]
             + [pltpu.SemaphoreType.DMA for _ in range(3 * _NSUB)])

  @functools.partial(
      pl.kernel,
      out_type=jax.ShapeDtypeStruct((_NC, _USERS, _D), jnp.float32),
      mesh=mesh,
      scratch_types=scratch,
  )
  def run(x_hbm, idx_hbm, sums_hbm, acc, *bufs):
    xb = bufs[:_NSUB]
    ib = bufs[_NSUB:2 * _NSUB]
    sems = bufs[2 * _NSUB:]
    sxx = sems[:_NSUB]
    six = sems[_NSUB:2 * _NSUB]
    ssx = sems[2 * _NSUB:3 * _NSUB]
    c = lax.axis_index("c")
    s = lax.axis_index("s")
    base = (c * _NS + s) * _TPW
    ibase = part * _PTOK + base

    zf = jnp.zeros((16,), jnp.float32)

    def zrow(r, carry):
      for k in range(_D // 16):
        xb[0][r, pl.ds(16 * k, 16)] = zf
      return carry
    lax.fori_loop(0, _SUB, zrow, 0)
    for r in range(_RPT // _SUB):
      pltpu.sync_copy(xb[0], acc.at[pl.ds(s * _RPT + r * _SUB, _SUB)])
    plsc.subcore_barrier()

    def body(i, carry):
      t0 = base + i * _CHUNK
      i0 = ibase + i * _CHUNK
      ins = []
      for j in range(_NSUB):
        @pl.when(i > 0)
        def _(j=j):
          pltpu.make_async_copy(xb[j], acc.at[ib[j]], ssx[j]).wait()
        hi = pltpu.async_copy(idx_hbm.at[pl.ds(i0 + j * _SUB, _SUB)],
                              ib[j], six[j])
        hx = pltpu.async_copy(x_hbm.at[pl.ds(t0 + j * _SUB, _SUB)], xb[j],
                              sxx[j])
        ins.append((hi, hx))
      for j in range(_NSUB):
        hi, hx = ins[j]
        hi.wait()
        hx.wait()
        pltpu.async_copy(xb[j], acc.at[ib[j]], ssx[j], add=True)
      return carry

    lax.fori_loop(0, _NCH, body, 0)
    for j in range(_NSUB):
      pltpu.make_async_copy(xb[j], acc.at[ib[j]], ssx[j]).wait()
    plsc.subcore_barrier()
    pltpu.sync_copy(acc.at[pl.ds(s * _RPT, _RPT)],
                    sums_hbm.at[c, pl.ds(s * _RPT, _RPT)])

  return run(x_part, idx_full)


def _finalize(parts, cnts):
  def body(*refs):
    o_ref = refs[-1]
    q = refs[_PARTS][...]
    tot = None
    for k in range(_PARTS):
      p = refs[k][...]
      for i in range(_NC):
        tot = p[i] if tot is None else tot + p[i]
    cnt = q[0, :, 0:1] + q[1, :, 0:1]
    val = tot / cnt
    o_ref[...] = jnp.concatenate(
        [val, jnp.zeros((_USERS, 1), jnp.float32)], axis=1)

  return pl.pallas_call(
      body,
      out_shape=jax.ShapeDtypeStruct((_USERS, _D + 1), jnp.float32),
  )(*parts, cnts)


@jax.jit
def kernel(x_hist, batch_hist):
  idx = batch_hist.astype(jnp.int32)
  cnts = _sc_counts(idx)
  parts = [
      _sc_part_sums(x_hist[k * _PTOK:(k + 1) * _PTOK], idx, k)
      for k in range(_PARTS)
  ]
  return _finalize(parts, cnts)


# speedup vs baseline: 1.4234x; 1.3604x over previous
import functools

import jax
import jax.numpy as jnp
from jax import lax
from jax.experimental import pallas as pl
from jax.experimental.pallas import tpu as pltpu
from jax.experimental.pallas import tpu_sc as plsc

_TOKENS = 409600
_PARTS = 5
_PTOK = _TOKENS // _PARTS
_D = 64
_USERS = 4096
_CW = 16
_NC = 2
_NS = 16
_NW = _NC * _NS
_TPW = _PTOK // _NW
_SUB = 128
_NSUB = 5
_CHUNK = _SUB * _NSUB
_NCH = _TPW // _CHUNK
_RPT = _USERS // _NS
_ITPW = _TOKENS // _NW
_INCH = _ITPW // _CHUNK


def _sc_counts(idx_full):
  mesh = plsc.VectorSubcoreMesh(core_axis_name="c", subcore_axis_name="s",
                                num_cores=_NC, num_subcores=_NS)
  scratch = ([pltpu.VMEM_SHARED((_USERS, _CW), jnp.float32)]
             + [pltpu.VMEM((_SUB,), jnp.int32) for _ in range(_NSUB)]
             + [pltpu.VMEM((_SUB, _CW), jnp.float32)]
             + [pltpu.SemaphoreType.DMA for _ in range(2 * _NSUB)])

  @functools.partial(
      pl.kernel,
      out_type=jax.ShapeDtypeStruct((_NC, _USERS, _CW), jnp.float32),
      mesh=mesh,
      scratch_types=scratch,
  )
  def run(idx_hbm, cnts_hbm, accc, *bufs):
    ib = bufs[:_NSUB]
    ones = bufs[_NSUB]
    sems = bufs[_NSUB + 1:]
    six = sems[:_NSUB]
    ssc = sems[_NSUB:]
    c = lax.axis_index("c")
    s = lax.axis_index("s")
    base = (c * _NS + s) * _ITPW

    zf = jnp.zeros((16,), jnp.float32)
    onesv = jnp.full((16,), 1.0, jnp.float32)

    def zcrow(r, carry):
      ones[r, pl.ds(0, 16)] = zf
      return carry
    lax.fori_loop(0, _SUB, zcrow, 0)
    for r in range(_RPT // _SUB):
      pltpu.sync_copy(ones, accc.at[pl.ds(s * _RPT + r * _SUB, _SUB)])
    def orow(r, carry):
      ones[r, pl.ds(0, 16)] = onesv
      return carry
    lax.fori_loop(0, _SUB, orow, 0)
    plsc.subcore_barrier()

    def body(i, carry):
      t0 = base + i * _CHUNK
      ins = []
      for j in range(_NSUB):
        @pl.when(i > 0)
        def _(j=j):
          pltpu.make_async_copy(ones, accc.at[ib[j]], ssc[j]).wait()
        ins.append(pltpu.async_copy(idx_hbm.at[pl.ds(t0 + j * _SUB, _SUB)],
                                    ib[j], six[j]))
      for j in range(_NSUB):
        ins[j].wait()
        pltpu.async_copy(ones, accc.at[ib[j]], ssc[j], add=True)
      return carry

    lax.fori_loop(0, _INCH, body, 0)
    for j in range(_NSUB):
      pltpu.make_async_copy(ones, accc.at[ib[j]], ssc[j]).wait()
    plsc.subcore_barrier()
    pltpu.sync_copy(accc.at[pl.ds(s * _RPT, _RPT)],
                    cnts_hbm.at[c, pl.ds(s * _RPT, _RPT)])

  return run(idx_full)


def _sc_part_sums(x_part, idx_full, part, cnts_dep):
  mesh = plsc.VectorSubcoreMesh(core_axis_name="c", subcore_axis_name="s",
                                num_cores=_NC, num_subcores=_NS)

  scratch = ([pltpu.VMEM_SHARED((_USERS, _D), jnp.float32)]
             + [pltpu.VMEM((_SUB, _D), jnp.float32) for _ in range(_NSUB)]
             + [pltpu.VMEM((_SUB,), jnp.int32) for _ in range(_NSUB)]
             + [pltpu.SemaphoreType.DMA for _ in range(3 * _NSUB)])

  @functools.partial(
      pl.kernel,
      out_type=jax.ShapeDtypeStruct((_NC, _USERS, _D), jnp.float32),
      mesh=mesh,
      scratch_types=scratch,
  )
  def run(x_hbm, idx_hbm, dep_hbm, sums_hbm, acc, *bufs):
    xb = bufs[:_NSUB]
    ib = bufs[_NSUB:2 * _NSUB]
    sems = bufs[2 * _NSUB:]
    sxx = sems[:_NSUB]
    six = sems[_NSUB:2 * _NSUB]
    ssx = sems[2 * _NSUB:3 * _NSUB]
    c = lax.axis_index("c")
    s = lax.axis_index("s")
    base = (c * _NS + s) * _TPW
    ibase = part * _PTOK + base

    zf = jnp.zeros((16,), jnp.float32)

    def zrow(r, carry):
      for k in range(_D // 16):
        xb[0][r, pl.ds(16 * k, 16)] = zf
      return carry
    lax.fori_loop(0, _SUB, zrow, 0)
    for r in range(_RPT // _SUB):
      pltpu.sync_copy(xb[0], acc.at[pl.ds(s * _RPT + r * _SUB, _SUB)])
    plsc.subcore_barrier()

    def body(i, carry):
      t0 = base + i * _CHUNK
      i0 = ibase + i * _CHUNK
      ins = []
      for j in range(_NSUB):
        @pl.when(i > 0)
        def _(j=j):
          pltpu.make_async_copy(xb[j], acc.at[ib[j]], ssx[j]).wait()
        hi = pltpu.async_copy(idx_hbm.at[pl.ds(i0 + j * _SUB, _SUB)],
                              ib[j], six[j])
        hx = pltpu.async_copy(x_hbm.at[pl.ds(t0 + j * _SUB, _SUB)], xb[j],
                              sxx[j])
        ins.append((hi, hx))
      for j in range(_NSUB):
        hi, hx = ins[j]
        hi.wait()
        hx.wait()
        pltpu.async_copy(xb[j], acc.at[ib[j]], ssx[j], add=True)
      return carry

    lax.fori_loop(0, _NCH, body, 0)
    for j in range(_NSUB):
      pltpu.make_async_copy(xb[j], acc.at[ib[j]], ssx[j]).wait()
    plsc.subcore_barrier()
    pltpu.sync_copy(acc.at[pl.ds(s * _RPT, _RPT)],
                    sums_hbm.at[c, pl.ds(s * _RPT, _RPT)])

  return run(x_part, idx_full, cnts_dep)


_RB = 4096


def _repack_part(xt, part):
  nb = _PTOK // _RB

  def body(x_ref, o_ref):
    o_ref[...] = x_ref[...].T

  return pl.pallas_call(
      body,
      grid=(nb,),
      in_specs=[pl.BlockSpec((_D, _RB),
                             lambda b, part=part: (0, part * nb + b))],
      out_specs=pl.BlockSpec((_RB, _D), lambda b: (b, 0)),
      out_shape=jax.ShapeDtypeStruct((_PTOK, _D), jnp.float32),
  )(xt)


def _finalize(parts, cnts):
  def body(*refs):
    o_ref = refs[-1]
    q = refs[_PARTS][...]
    tot = None
    for k in range(_PARTS):
      p = refs[k][...]
      for i in range(_NC):
        tot = p[i] if tot is None else tot + p[i]
    cnt = q[0, :, 0:1] + q[1, :, 0:1]
    val = tot / cnt
    o_ref[...] = jnp.concatenate(
        [val, jnp.zeros((_USERS, 1), jnp.float32)], axis=1)

  return pl.pallas_call(
      body,
      out_shape=jax.ShapeDtypeStruct((_USERS, _D + 1), jnp.float32),
  )(*parts, cnts)


@jax.jit
def kernel(x_hist, batch_hist):
  idx = batch_hist.astype(jnp.int32)
  xt = jnp.swapaxes(x_hist, 0, 1)
  cnts = _sc_counts(idx)
  parts = [
      _sc_part_sums(_repack_part(xt, k), idx, k, cnts)
      for k in range(_PARTS)
  ]
  return _finalize(parts, cnts)


# speedup vs baseline: 1.4425x; 1.0134x over previous
import functools

import jax
import jax.numpy as jnp
from jax import lax
from jax.experimental import pallas as pl
from jax.experimental.pallas import tpu as pltpu
from jax.experimental.pallas import tpu_sc as plsc

_TOKENS = 409600
_PARTS = 4
_PTOK = _TOKENS // _PARTS
_D = 64
_USERS = 4096
_CW = 16
_NC = 2
_NS = 16
_NW = _NC * _NS
_TPW = _PTOK // _NW
_SUB = 128
_NSUB = 5
_CHUNK = _SUB * _NSUB
_NCH = _TPW // _CHUNK
_RPT = _USERS // _NS
_ITPW = _TOKENS // _NW
_INCH = _ITPW // _CHUNK


def _sc_counts(idx_full):
  mesh = plsc.VectorSubcoreMesh(core_axis_name="c", subcore_axis_name="s",
                                num_cores=_NC, num_subcores=_NS)
  scratch = ([pltpu.VMEM_SHARED((_USERS, _CW), jnp.float32)]
             + [pltpu.VMEM((_SUB,), jnp.int32) for _ in range(_NSUB)]
             + [pltpu.VMEM((_SUB, _CW), jnp.float32)]
             + [pltpu.SemaphoreType.DMA for _ in range(2 * _NSUB)])

  @functools.partial(
      pl.kernel,
      out_type=jax.ShapeDtypeStruct((_NC, _USERS, _CW), jnp.float32),
      mesh=mesh,
      scratch_types=scratch,
  )
  def run(idx_hbm, cnts_hbm, accc, *bufs):
    ib = bufs[:_NSUB]
    ones = bufs[_NSUB]
    sems = bufs[_NSUB + 1:]
    six = sems[:_NSUB]
    ssc = sems[_NSUB:]
    c = lax.axis_index("c")
    s = lax.axis_index("s")
    base = (c * _NS + s) * _ITPW

    zf = jnp.zeros((16,), jnp.float32)
    onesv = jnp.full((16,), 1.0, jnp.float32)

    def zcrow(r, carry):
      ones[r, pl.ds(0, 16)] = zf
      return carry
    lax.fori_loop(0, _SUB, zcrow, 0)
    for r in range(_RPT // _SUB):
      pltpu.sync_copy(ones, accc.at[pl.ds(s * _RPT + r * _SUB, _SUB)])
    def orow(r, carry):
      ones[r, pl.ds(0, 16)] = onesv
      return carry
    lax.fori_loop(0, _SUB, orow, 0)
    plsc.subcore_barrier()

    def body(i, carry):
      t0 = base + i * _CHUNK
      ins = []
      for j in range(_NSUB):
        @pl.when(i > 0)
        def _(j=j):
          pltpu.make_async_copy(ones, accc.at[ib[j]], ssc[j]).wait()
        ins.append(pltpu.async_copy(idx_hbm.at[pl.ds(t0 + j * _SUB, _SUB)],
                                    ib[j], six[j]))
      for j in range(_NSUB):
        ins[j].wait()
        pltpu.async_copy(ones, accc.at[ib[j]], ssc[j], add=True)
      return carry

    lax.fori_loop(0, _INCH, body, 0)
    for j in range(_NSUB):
      pltpu.make_async_copy(ones, accc.at[ib[j]], ssc[j]).wait()
    plsc.subcore_barrier()
    pltpu.sync_copy(accc.at[pl.ds(s * _RPT, _RPT)],
                    cnts_hbm.at[c, pl.ds(s * _RPT, _RPT)])

  return run(idx_full)


def _sc_part_sums(x_part, idx_full, part, cnts_dep):
  mesh = plsc.VectorSubcoreMesh(core_axis_name="c", subcore_axis_name="s",
                                num_cores=_NC, num_subcores=_NS)

  scratch = ([pltpu.VMEM_SHARED((_USERS, _D), jnp.float32)]
             + [pltpu.VMEM((_SUB, _D), jnp.float32) for _ in range(_NSUB)]
             + [pltpu.VMEM((_SUB,), jnp.int32) for _ in range(_NSUB)]
             + [pltpu.SemaphoreType.DMA for _ in range(3 * _NSUB)])

  @functools.partial(
      pl.kernel,
      out_type=jax.ShapeDtypeStruct((_NC, _USERS, _D), jnp.float32),
      mesh=mesh,
      scratch_types=scratch,
  )
  def run(x_hbm, idx_hbm, dep_hbm, sums_hbm, acc, *bufs):
    xb = bufs[:_NSUB]
    ib = bufs[_NSUB:2 * _NSUB]
    sems = bufs[2 * _NSUB:]
    sxx = sems[:_NSUB]
    six = sems[_NSUB:2 * _NSUB]
    ssx = sems[2 * _NSUB:3 * _NSUB]
    c = lax.axis_index("c")
    s = lax.axis_index("s")
    base = (c * _NS + s) * _TPW
    ibase = part * _PTOK + base

    zf = jnp.zeros((16,), jnp.float32)

    def zrow(r, carry):
      for k in range(_D // 16):
        xb[0][r, pl.ds(16 * k, 16)] = zf
      return carry
    lax.fori_loop(0, _SUB, zrow, 0)
    for r in range(_RPT // _SUB):
      pltpu.sync_copy(xb[0], acc.at[pl.ds(s * _RPT + r * _SUB, _SUB)])
    plsc.subcore_barrier()

    def body(i, carry):
      t0 = base + i * _CHUNK
      i0 = ibase + i * _CHUNK
      ins = []
      for j in range(_NSUB):
        @pl.when(i > 0)
        def _(j=j):
          pltpu.make_async_copy(xb[j], acc.at[ib[j]], ssx[j]).wait()
        hi = pltpu.async_copy(idx_hbm.at[pl.ds(i0 + j * _SUB, _SUB)],
                              ib[j], six[j])
        hx = pltpu.async_copy(x_hbm.at[pl.ds(t0 + j * _SUB, _SUB)], xb[j],
                              sxx[j])
        ins.append((hi, hx))
      for j in range(_NSUB):
        hi, hx = ins[j]
        hi.wait()
        hx.wait()
        pltpu.async_copy(xb[j], acc.at[ib[j]], ssx[j], add=True)
      return carry

    lax.fori_loop(0, _NCH, body, 0)
    for j in range(_NSUB):
      pltpu.make_async_copy(xb[j], acc.at[ib[j]], ssx[j]).wait()
    plsc.subcore_barrier()
    pltpu.sync_copy(acc.at[pl.ds(s * _RPT, _RPT)],
                    sums_hbm.at[c, pl.ds(s * _RPT, _RPT)])

  return run(x_part, idx_full, cnts_dep)


_RB = 8192


def _repack_part(xt, part):
  nb = _PTOK // _RB

  def body(x_ref, o_ref):
    o_ref[...] = x_ref[...].T

  return pl.pallas_call(
      body,
      grid=(nb,),
      in_specs=[pl.BlockSpec((_D, _RB),
                             lambda b, part=part: (0, part * nb + b))],
      out_specs=pl.BlockSpec((_RB, _D), lambda b: (b, 0)),
      out_shape=jax.ShapeDtypeStruct((_PTOK, _D), jnp.float32),
  )(xt)


def _finalize(parts, cnts):
  def body(*refs):
    o_ref = refs[-1]
    q = refs[_PARTS][...]
    tot = None
    for k in range(_PARTS):
      p = refs[k][...]
      for i in range(_NC):
        tot = p[i] if tot is None else tot + p[i]
    cnt = q[0, :, 0:1] + q[1, :, 0:1]
    val = tot / cnt
    o_ref[...] = jnp.concatenate(
        [val, jnp.zeros((_USERS, 1), jnp.float32)], axis=1)

  return pl.pallas_call(
      body,
      out_shape=jax.ShapeDtypeStruct((_USERS, _D + 1), jnp.float32),
  )(*parts, cnts)


@jax.jit
def kernel(x_hist, batch_hist):
  idx = batch_hist.astype(jnp.int32)
  xt = jnp.swapaxes(x_hist, 0, 1)
  cnts = _sc_counts(idx)
  parts = [
      _sc_part_sums(_repack_part(xt, k), idx, k, cnts)
      for k in range(_PARTS)
  ]
  return _finalize(parts, cnts)


# speedup vs baseline: 1.4533x; 1.0075x over previous
import functools

import jax
import jax.numpy as jnp
from jax import lax
from jax.experimental import pallas as pl
from jax.experimental.pallas import tpu as pltpu
from jax.experimental.pallas import tpu_sc as plsc

_TOKENS = 409600
_PARTS = 4
_PTOK = _TOKENS // _PARTS
_D = 64
_USERS = 4096
_CW = 16
_NC = 2
_NS = 16
_NW = _NC * _NS
_TPW = _PTOK // _NW
_SUB = 128
_NSUB = 5
_CHUNK = _SUB * _NSUB
_NCH = _TPW // _CHUNK
_RPT = _USERS // _NS
_ITPW = _TOKENS // _NW
_INCH = _ITPW // _CHUNK


def _sc_counts(idx_full):
  mesh = plsc.VectorSubcoreMesh(core_axis_name="c", subcore_axis_name="s",
                                num_cores=_NC, num_subcores=_NS)
  scratch = ([pltpu.VMEM_SHARED((_USERS, _CW), jnp.float32)]
             + [pltpu.VMEM((_SUB,), jnp.int32) for _ in range(_NSUB)]
             + [pltpu.VMEM((_SUB, _CW), jnp.float32)]
             + [pltpu.SemaphoreType.DMA for _ in range(2 * _NSUB)])

  @functools.partial(
      pl.kernel,
      out_type=jax.ShapeDtypeStruct((_NC, _USERS, _CW), jnp.float32),
      mesh=mesh,
      scratch_types=scratch,
  )
  def run(idx_hbm, cnts_hbm, accc, *bufs):
    ib = bufs[:_NSUB]
    ones = bufs[_NSUB]
    sems = bufs[_NSUB + 1:]
    six = sems[:_NSUB]
    ssc = sems[_NSUB:]
    c = lax.axis_index("c")
    s = lax.axis_index("s")
    base = (c * _NS + s) * _ITPW

    zf = jnp.zeros((16,), jnp.float32)
    onesv = jnp.full((16,), 1.0, jnp.float32)

    def zcrow(r, carry):
      ones[r, pl.ds(0, 16)] = zf
      return carry
    lax.fori_loop(0, _SUB, zcrow, 0)
    for r in range(_RPT // _SUB):
      pltpu.sync_copy(ones, accc.at[pl.ds(s * _RPT + r * _SUB, _SUB)])
    def orow(r, carry):
      ones[r, pl.ds(0, 16)] = onesv
      return carry
    lax.fori_loop(0, _SUB, orow, 0)
    plsc.subcore_barrier()

    def body(i, carry):
      t0 = base + i * _CHUNK
      ins = []
      for j in range(_NSUB):
        @pl.when(i > 0)
        def _(j=j):
          pltpu.make_async_copy(ones, accc.at[ib[j]], ssc[j]).wait()
        ins.append(pltpu.async_copy(idx_hbm.at[pl.ds(t0 + j * _SUB, _SUB)],
                                    ib[j], six[j]))
      for j in range(_NSUB):
        ins[j].wait()
        pltpu.async_copy(ones, accc.at[ib[j]], ssc[j], add=True)
      return carry

    lax.fori_loop(0, _INCH, body, 0)
    for j in range(_NSUB):
      pltpu.make_async_copy(ones, accc.at[ib[j]], ssc[j]).wait()
    plsc.subcore_barrier()
    pltpu.sync_copy(accc.at[pl.ds(s * _RPT, _RPT)],
                    cnts_hbm.at[c, pl.ds(s * _RPT, _RPT)])

  return run(idx_full)


def _sc_part_sums(x_part, idx_full, part, cnts_dep):
  mesh = plsc.VectorSubcoreMesh(core_axis_name="c", subcore_axis_name="s",
                                num_cores=_NC, num_subcores=_NS)

  scratch = ([pltpu.VMEM_SHARED((_USERS, _D), jnp.float32)]
             + [pltpu.VMEM((_SUB, _D), jnp.float32) for _ in range(_NSUB)]
             + [pltpu.VMEM((_SUB,), jnp.int32) for _ in range(_NSUB)]
             + [pltpu.SemaphoreType.DMA for _ in range(3 * _NSUB)])

  @functools.partial(
      pl.kernel,
      out_type=jax.ShapeDtypeStruct((_NC, _USERS, _D), jnp.float32),
      mesh=mesh,
      scratch_types=scratch,
  )
  def run(x_hbm, idx_hbm, dep_hbm, sums_hbm, acc, *bufs):
    xb = bufs[:_NSUB]
    ib = bufs[_NSUB:2 * _NSUB]
    sems = bufs[2 * _NSUB:]
    sxx = sems[:_NSUB]
    six = sems[_NSUB:2 * _NSUB]
    ssx = sems[2 * _NSUB:3 * _NSUB]
    c = lax.axis_index("c")
    s = lax.axis_index("s")
    base = (c * _NS + s) * _TPW
    ibase = part * _PTOK + base

    zf = jnp.zeros((16,), jnp.float32)

    def zrow(r, carry):
      for k in range(_D // 16):
        xb[0][r, pl.ds(16 * k, 16)] = zf
      return carry
    lax.fori_loop(0, _SUB, zrow, 0)
    for r in range(_RPT // _SUB):
      pltpu.sync_copy(xb[0], acc.at[pl.ds(s * _RPT + r * _SUB, _SUB)])
    plsc.subcore_barrier()

    def body(i, carry):
      t0 = base + i * _CHUNK
      i0 = ibase + i * _CHUNK
      ins = []
      for j in range(_NSUB):
        @pl.when(i > 0)
        def _(j=j):
          pltpu.make_async_copy(xb[j], acc.at[ib[j]], ssx[j]).wait()
        hi = pltpu.async_copy(idx_hbm.at[pl.ds(i0 + j * _SUB, _SUB)],
                              ib[j], six[j])
        hx = pltpu.async_copy(x_hbm.at[pl.ds(t0 + j * _SUB, _SUB)], xb[j],
                              sxx[j])
        ins.append((hi, hx))
      for j in range(_NSUB):
        hi, hx = ins[j]
        hi.wait()
        hx.wait()
        pltpu.async_copy(xb[j], acc.at[ib[j]], ssx[j], add=True)
      return carry

    lax.fori_loop(0, _NCH, body, 0)
    for j in range(_NSUB):
      pltpu.make_async_copy(xb[j], acc.at[ib[j]], ssx[j]).wait()
    plsc.subcore_barrier()
    pltpu.sync_copy(acc.at[pl.ds(s * _RPT, _RPT)],
                    sums_hbm.at[c, pl.ds(s * _RPT, _RPT)])

  return run(x_part, idx_full, cnts_dep)


_RB = 4096


def _repack_part(xt, part):
  nb = _PTOK // _RB

  def body(x_ref, o_ref):
    o_ref[...] = x_ref[...].T

  return pl.pallas_call(
      body,
      grid=(nb,),
      in_specs=[pl.BlockSpec((_D, _RB),
                             lambda b, part=part: (0, part * nb + b))],
      out_specs=pl.BlockSpec((_RB, _D), lambda b: (b, 0)),
      out_shape=jax.ShapeDtypeStruct((_PTOK, _D), jnp.float32),
  )(xt)


def _finalize(parts, cnts):
  def body(*refs):
    o_ref = refs[-1]
    q = refs[_PARTS][...]
    tot = None
    for k in range(_PARTS):
      p = refs[k][...]
      for i in range(_NC):
        tot = p[i] if tot is None else tot + p[i]
    cnt = q[0, :, 0:1] + q[1, :, 0:1]
    val = tot / cnt
    o_ref[...] = jnp.concatenate(
        [val, jnp.zeros((_USERS, 1), jnp.float32)], axis=1)

  return pl.pallas_call(
      body,
      out_shape=jax.ShapeDtypeStruct((_USERS, _D + 1), jnp.float32),
  )(*parts, cnts)


@jax.jit
def kernel(x_hist, batch_hist):
  idx = batch_hist.astype(jnp.int32)
  xt = jnp.swapaxes(x_hist, 0, 1)
  cnts = _sc_counts(idx)
  parts = [
      _sc_part_sums(_repack_part(xt, k), idx, k, cnts)
      for k in range(_PARTS)
  ]
  return _finalize(parts, cnts)
